# Initial kernel scaffold; baseline (speedup 1.0000x reference)
import jax
import jax.numpy as jnp
from jax.experimental import pallas as pl


def kernel(x, edge_index, W_self1, W_neigh1, b1, W_self2, W_neigh2, b2):
    raise NotImplementedError("write your pallas kernel here")


# speedup vs baseline: 5.9162x; 5.9162x over previous
import functools

import jax
import jax.numpy as jnp
from jax import lax
from jax.experimental import pallas as pl
from jax.experimental.pallas import tpu as pltpu
from jax.experimental.pallas import tpu_sc as plsc

N = 10000
E = 320000
D_IN = 128
D_H = 64

NC = 2
NS = 16
NW = NC * NS
EPW = E // NW
K = 80
CHUNKS = EPW // K
N_PAD = 10240
NPT = N_PAD // NS

_MESH = plsc.VectorSubcoreMesh(core_axis_name="c", subcore_axis_name="s")
_SC_PARAMS = pltpu.CompilerParams(use_tc_tiling_on_sc=False)


def _sc_agg_body(with_deg, *refs):
    if with_deg:
        (y_hbm, src_hbm, dst_hbm, z64, z16, ones_hbm, agg_out, deg_out,
         idx_s, idx_d, rows, ones_v, acc_sh, deg_sh, sem) = refs
    else:
        (y_hbm, src_hbm, dst_hbm, z64, agg_out,
         idx_s, idx_d, rows, acc_sh, sem) = refs

    c = lax.axis_index("c")
    s = lax.axis_index("s")
    nbase = s * NPT

    pltpu.sync_copy(z64.at[pl.ds(nbase, NPT)], acc_sh.at[pl.ds(nbase, NPT)])
    if with_deg:
        pltpu.sync_copy(z16.at[pl.ds(nbase, NPT)], deg_sh.at[pl.ds(nbase, NPT)])
        pltpu.sync_copy(ones_hbm, ones_v)
    plsc.subcore_barrier()

    ebase = (c * NS + s) * EPW

    def chunk(i, carry):
        off = ebase + i * K
        pltpu.sync_copy(src_hbm.at[pl.ds(off, K)], idx_s)
        pltpu.sync_copy(dst_hbm.at[pl.ds(off, K)], idx_d)
        pltpu.async_copy(y_hbm.at[idx_s], rows, sem).wait()
        pltpu.sync_copy(rows, acc_sh.at[idx_d], add=True)
        if with_deg:
            pltpu.sync_copy(ones_v, deg_sh.at[idx_d], add=True)
        return carry

    lax.fori_loop(0, CHUNKS, chunk, 0)
    plsc.subcore_barrier()

    pltpu.sync_copy(acc_sh.at[pl.ds(nbase, NPT)],
                    agg_out.at[c, pl.ds(nbase, NPT)])
    if with_deg:
        pltpu.sync_copy(deg_sh.at[pl.ds(nbase, NPT)],
                        deg_out.at[c, pl.ds(nbase, NPT)])


_sc_agg_deg = pl.kernel(
    functools.partial(_sc_agg_body, True),
    out_type=(jax.ShapeDtypeStruct((NC, N_PAD, D_H), jnp.float32),
              jax.ShapeDtypeStruct((NC, N_PAD, 16), jnp.float32)),
    mesh=_MESH,
    compiler_params=_SC_PARAMS,
    scratch_types=[
        pltpu.VMEM((K,), jnp.int32),
        pltpu.VMEM((K,), jnp.int32),
        pltpu.VMEM((K, D_H), jnp.float32),
        pltpu.VMEM((K, 16), jnp.float32),
        pltpu.VMEM_SHARED((N_PAD, D_H), jnp.float32),
        pltpu.VMEM_SHARED((N_PAD, 16), jnp.float32),
        pltpu.SemaphoreType.DMA,
    ],
)

_sc_agg = pl.kernel(
    functools.partial(_sc_agg_body, False),
    out_type=jax.ShapeDtypeStruct((NC, N_PAD, D_H), jnp.float32),
    mesh=_MESH,
    compiler_params=_SC_PARAMS,
    scratch_types=[
        pltpu.VMEM((K,), jnp.int32),
        pltpu.VMEM((K,), jnp.int32),
        pltpu.VMEM((K, D_H), jnp.float32),
        pltpu.VMEM_SHARED((N_PAD, D_H), jnp.float32),
        pltpu.SemaphoreType.DMA,
    ],
)


_BN = 1000


def _tc1_body(x_ref, ws_ref, wn_ref, b_ref, s_ref, y_ref):
    x = x_ref[...]
    s_ref[...] = jnp.dot(x, ws_ref[...], preferred_element_type=jnp.float32) + b_ref[...]
    y_ref[...] = jnp.dot(x, wn_ref[...], preferred_element_type=jnp.float32)


def _tc2_body(s1_ref, agg_ref, deg_ref, ws_ref, wn_ref, b_ref, s2_ref, y2_ref):
    agg = agg_ref[0] + agg_ref[1]
    deg = jnp.maximum(deg_ref[0][:, 0:1] + deg_ref[1][:, 0:1], 1.0)
    h = jnp.maximum(s1_ref[...] + agg / deg, 0.0)
    s2_ref[...] = jnp.dot(h, ws_ref[...], preferred_element_type=jnp.float32) + b_ref[...]
    y2_ref[...] = jnp.dot(h, wn_ref[...], preferred_element_type=jnp.float32)


def _tc3_body(s2_ref, agg_ref, deg_ref, out_ref):
    agg = agg_ref[0] + agg_ref[1]
    deg = jnp.maximum(deg_ref[0][:, 0:1] + deg_ref[1][:, 0:1], 1.0)
    out_ref[...] = s2_ref[...] + agg / deg


def _row_spec(d):
    return pl.BlockSpec((_BN, d), lambda i: (i, 0))


def _part_spec(d):
    return pl.BlockSpec((NC, _BN, d), lambda i: (0, i, 0))


def _full_spec(r, c):
    return pl.BlockSpec((r, c), lambda i: (0, 0))


_tc1 = pl.pallas_call(
    _tc1_body,
    grid=(N // _BN,),
    in_specs=[_row_spec(D_IN), _full_spec(D_IN, D_H), _full_spec(D_IN, D_H),
              pl.BlockSpec((D_H,), lambda i: (0,))],
    out_specs=[_row_spec(D_H), _row_spec(D_H)],
    out_shape=[jax.ShapeDtypeStruct((N, D_H), jnp.float32)] * 2,
)

_tc2 = pl.pallas_call(
    _tc2_body,
    grid=(N // _BN,),
    in_specs=[_row_spec(D_H), _part_spec(D_H), _part_spec(16),
              _full_spec(D_H, D_H), _full_spec(D_H, D_H),
              pl.BlockSpec((D_H,), lambda i: (0,))],
    out_specs=[_row_spec(D_H), _row_spec(D_H)],
    out_shape=[jax.ShapeDtypeStruct((N, D_H), jnp.float32)] * 2,
)

_tc3 = pl.pallas_call(
    _tc3_body,
    grid=(N // _BN,),
    in_specs=[_row_spec(D_H), _part_spec(D_H), _part_spec(16)],
    out_specs=_row_spec(D_H),
    out_shape=jax.ShapeDtypeStruct((N, D_H), jnp.float32),
)


def kernel(x, edge_index, W_self1, W_neigh1, b1, W_self2, W_neigh2, b2):
    src = edge_index[0].astype(jnp.int32)
    dst = edge_index[1].astype(jnp.int32)
    z64 = jnp.zeros((N_PAD, D_H), jnp.float32)
    z16 = jnp.zeros((N_PAD, 16), jnp.float32)
    ones = jnp.ones((K, 16), jnp.float32)

    s1, y1 = _tc1(x, W_self1, W_neigh1, b1)
    agg1, deg = _sc_agg_deg(y1, src, dst, z64, z16, ones)
    s2, y2 = _tc2(s1, agg1, deg, W_self2, W_neigh2, b2)
    agg2 = _sc_agg(y2, src, dst, z64)
    return _tc3(s2, agg2, deg)


# speedup vs baseline: 15.1609x; 2.5626x over previous
import functools

import jax
import jax.numpy as jnp
from jax import lax
from jax.experimental import pallas as pl
from jax.experimental.pallas import tpu as pltpu
from jax.experimental.pallas import tpu_sc as plsc

N = 10000
E = 320000
D_IN = 128
D_H = 64

NC = 2
NS = 16
NW = NC * NS
EPW = E // NW
K = 80
CHUNKS = EPW // K
NBUF = 5
N_PAD = 10240
NPT = N_PAD // NS

_MESH = plsc.VectorSubcoreMesh(core_axis_name="c", subcore_axis_name="s")
_SC_PARAMS = pltpu.CompilerParams(use_tc_tiling_on_sc=False)


def _sc_agg_body(with_deg, *refs):
    if with_deg:
        (y_hbm, src_hbm, dst_hbm, z64, z16, ones_hbm, agg_out, deg_out,
         idx_s, idx_d, ones_v, acc_sh, deg_sh) = refs[:13]
        rows = refs[13:13 + NBUF]
        gsem = refs[13 + NBUF:13 + 2 * NBUF]
        ssem = refs[13 + 2 * NBUF:13 + 3 * NBUF]
        dsem = refs[13 + 3 * NBUF:13 + 4 * NBUF]
    else:
        (y_hbm, src_hbm, dst_hbm, z64, agg_out,
         idx_s, idx_d, acc_sh) = refs[:8]
        rows = refs[8:8 + NBUF]
        gsem = refs[8 + NBUF:8 + 2 * NBUF]
        ssem = refs[8 + 2 * NBUF:8 + 3 * NBUF]

    c = lax.axis_index("c")
    s = lax.axis_index("s")
    nbase = s * NPT
    wid = c * NS + s

    pltpu.sync_copy(z64.at[pl.ds(nbase, NPT)], acc_sh.at[pl.ds(nbase, NPT)])
    pltpu.sync_copy(src_hbm.at[wid], idx_s)
    pltpu.sync_copy(dst_hbm.at[wid], idx_d)
    if with_deg:
        pltpu.sync_copy(z16.at[pl.ds(nbase, NPT)], deg_sh.at[pl.ds(nbase, NPT)])
        pltpu.sync_copy(ones_hbm, ones_v)
    plsc.subcore_barrier()

    def start_gather(b, row):
        pltpu.async_copy(y_hbm.at[idx_s.at[row]], rows[b], gsem[b])

    def wait_gather(b):
        pltpu.make_async_copy(y_hbm.at[idx_s.at[0]], rows[b], gsem[b]).wait()

    for b in range(NBUF):
        start_gather(b, b)

    def round_(j, carry):
        for b in range(NBUF):
            row = j * NBUF + b
            wait_gather(b)
            pltpu.async_copy(rows[b], acc_sh.at[idx_d.at[row]], ssem[b],
                             add=True)
            if with_deg:
                pltpu.async_copy(ones_v, deg_sh.at[idx_d.at[row]], dsem[b],
                                 add=True)
        for b in range(NBUF):
            pltpu.make_async_copy(rows[b], acc_sh.at[idx_d.at[0]],
                                  ssem[b]).wait()
            if with_deg:
                pltpu.make_async_copy(ones_v, deg_sh.at[idx_d.at[0]],
                                      dsem[b]).wait()
            nrow = jnp.minimum((j + 1) * NBUF + b, CHUNKS - 1)
            start_gather(b, nrow)
        return carry

    lax.fori_loop(0, CHUNKS // NBUF, round_, 0)
    for b in range(NBUF):
        wait_gather(b)
    plsc.subcore_barrier()

    pltpu.sync_copy(acc_sh.at[pl.ds(nbase, NPT)],
                    agg_out.at[c, pl.ds(nbase, NPT)])
    if with_deg:
        pltpu.sync_copy(deg_sh.at[pl.ds(nbase, NPT)],
                        deg_out.at[c, pl.ds(nbase, NPT)])


_sc_agg_deg = pl.kernel(
    functools.partial(_sc_agg_body, True),
    out_type=(jax.ShapeDtypeStruct((NC, N_PAD, D_H), jnp.float32),
              jax.ShapeDtypeStruct((NC, N_PAD, 16), jnp.float32)),
    mesh=_MESH,
    compiler_params=_SC_PARAMS,
    scratch_types=(
        [pltpu.VMEM((CHUNKS, K), jnp.int32),
         pltpu.VMEM((CHUNKS, K), jnp.int32),
         pltpu.VMEM((K, 16), jnp.float32),
         pltpu.VMEM_SHARED((N_PAD, D_H), jnp.float32),
         pltpu.VMEM_SHARED((N_PAD, 16), jnp.float32)]
        + [pltpu.VMEM((K, D_H), jnp.float32)] * NBUF
        + [pltpu.SemaphoreType.DMA] * (3 * NBUF)
    ),
)

_sc_agg = pl.kernel(
    functools.partial(_sc_agg_body, False),
    out_type=jax.ShapeDtypeStruct((NC, N_PAD, D_H), jnp.float32),
    mesh=_MESH,
    compiler_params=_SC_PARAMS,
    scratch_types=(
        [pltpu.VMEM((CHUNKS, K), jnp.int32),
         pltpu.VMEM((CHUNKS, K), jnp.int32),
         pltpu.VMEM_SHARED((N_PAD, D_H), jnp.float32)]
        + [pltpu.VMEM((K, D_H), jnp.float32)] * NBUF
        + [pltpu.SemaphoreType.DMA] * (2 * NBUF)
    ),
)


_BN = 1000


def _tc1_body(x_ref, ws_ref, wn_ref, b_ref, s_ref, y_ref):
    x = x_ref[...]
    s_ref[...] = jnp.dot(x, ws_ref[...], preferred_element_type=jnp.float32) + b_ref[...]
    y_ref[...] = jnp.dot(x, wn_ref[...], preferred_element_type=jnp.float32)


def _tc2_body(s1_ref, agg_ref, deg_ref, ws_ref, wn_ref, b_ref, s2_ref, y2_ref):
    agg = agg_ref[0] + agg_ref[1]
    deg = jnp.maximum(deg_ref[0][:, 0:1] + deg_ref[1][:, 0:1], 1.0)
    h = jnp.maximum(s1_ref[...] + agg / deg, 0.0)
    s2_ref[...] = jnp.dot(h, ws_ref[...], preferred_element_type=jnp.float32) + b_ref[...]
    y2_ref[...] = jnp.dot(h, wn_ref[...], preferred_element_type=jnp.float32)


def _tc3_body(s2_ref, agg_ref, deg_ref, out_ref):
    agg = agg_ref[0] + agg_ref[1]
    deg = jnp.maximum(deg_ref[0][:, 0:1] + deg_ref[1][:, 0:1], 1.0)
    out_ref[...] = s2_ref[...] + agg / deg


def _row_spec(d):
    return pl.BlockSpec((_BN, d), lambda i: (i, 0))


def _part_spec(d):
    return pl.BlockSpec((NC, _BN, d), lambda i: (0, i, 0))


def _full_spec(r, c):
    return pl.BlockSpec((r, c), lambda i: (0, 0))


_tc1 = pl.pallas_call(
    _tc1_body,
    grid=(N // _BN,),
    in_specs=[_row_spec(D_IN), _full_spec(D_IN, D_H), _full_spec(D_IN, D_H),
              pl.BlockSpec((D_H,), lambda i: (0,))],
    out_specs=[_row_spec(D_H), _row_spec(D_H)],
    out_shape=[jax.ShapeDtypeStruct((N, D_H), jnp.float32)] * 2,
)

_tc2 = pl.pallas_call(
    _tc2_body,
    grid=(N // _BN,),
    in_specs=[_row_spec(D_H), _part_spec(D_H), _part_spec(16),
              _full_spec(D_H, D_H), _full_spec(D_H, D_H),
              pl.BlockSpec((D_H,), lambda i: (0,))],
    out_specs=[_row_spec(D_H), _row_spec(D_H)],
    out_shape=[jax.ShapeDtypeStruct((N, D_H), jnp.float32)] * 2,
)

_tc3 = pl.pallas_call(
    _tc3_body,
    grid=(N // _BN,),
    in_specs=[_row_spec(D_H), _part_spec(D_H), _part_spec(16)],
    out_specs=_row_spec(D_H),
    out_shape=jax.ShapeDtypeStruct((N, D_H), jnp.float32),
)


def kernel(x, edge_index, W_self1, W_neigh1, b1, W_self2, W_neigh2, b2):
    src = edge_index[0].astype(jnp.int32).reshape(NW, CHUNKS, K)
    dst = edge_index[1].astype(jnp.int32).reshape(NW, CHUNKS, K)
    z64 = jnp.zeros((N_PAD, D_H), jnp.float32)
    z16 = jnp.zeros((N_PAD, 16), jnp.float32)
    ones = jnp.ones((K, 16), jnp.float32)

    s1, y1 = _tc1(x, W_self1, W_neigh1, b1)
    agg1, deg = _sc_agg_deg(y1, src, dst, z64, z16, ones)
    s2, y2 = _tc2(s1, agg1, deg, W_self2, W_neigh2, b2)
    agg2 = _sc_agg(y2, src, dst, z64)
    return _tc3(s2, agg2, deg)


# speedup vs baseline: 15.2765x; 1.0076x over previous
import functools

import jax
import jax.numpy as jnp
from jax import lax
from jax.experimental import pallas as pl
from jax.experimental.pallas import tpu as pltpu
from jax.experimental.pallas import tpu_sc as plsc

N = 10000
E = 320000
D_IN = 128
D_H = 64

NC = 2
NS = 16
NW = NC * NS
EPW = E // NW
K = 80
CHUNKS = EPW // K
NBUF = 5
N_PAD = 10240
NPT = N_PAD // NS

_MESH = plsc.VectorSubcoreMesh(core_axis_name="c", subcore_axis_name="s")
_SC_PARAMS = pltpu.CompilerParams(use_tc_tiling_on_sc=False)


def _sc_agg_body(with_deg, *refs):
    if with_deg:
        (y_hbm, src_hbm, dst_hbm, z64, z16, ones_hbm, agg_out, deg_out,
         idx_s, idx_d, ones_v, acc_sh, deg_sh) = refs[:13]
        rows = refs[13:13 + NBUF]
        gsem = refs[13 + NBUF:13 + 2 * NBUF]
        ssem = refs[13 + 2 * NBUF:13 + 3 * NBUF]
        dsem = refs[13 + 3 * NBUF:13 + 4 * NBUF]
    else:
        (y_hbm, src_hbm, dst_hbm, z64, agg_out,
         idx_s, idx_d, acc_sh) = refs[:8]
        rows = refs[8:8 + NBUF]
        gsem = refs[8 + NBUF:8 + 2 * NBUF]
        ssem = refs[8 + 2 * NBUF:8 + 3 * NBUF]

    c = lax.axis_index("c")
    s = lax.axis_index("s")
    nbase = s * NPT
    wid = c * NS + s

    pltpu.sync_copy(z64.at[pl.ds(nbase, NPT)], acc_sh.at[pl.ds(nbase, NPT)])
    pltpu.sync_copy(src_hbm.at[wid], idx_s)
    pltpu.sync_copy(dst_hbm.at[wid], idx_d)
    if with_deg:
        pltpu.sync_copy(z16.at[pl.ds(nbase, NPT)], deg_sh.at[pl.ds(nbase, NPT)])
        pltpu.sync_copy(ones_hbm, ones_v)
    plsc.subcore_barrier()

    def start_gather(b, row):
        pltpu.async_copy(y_hbm.at[idx_s.at[row]], rows[b], gsem[b])

    def wait_gather(b):
        pltpu.make_async_copy(y_hbm.at[idx_s.at[0]], rows[b], gsem[b]).wait()

    for b in range(NBUF):
        start_gather(b, b)

    def round_(j, carry):
        for b in range(NBUF):
            row = j * NBUF + b
            wait_gather(b)
            pltpu.async_copy(rows[b], acc_sh.at[idx_d.at[row]], ssem[b],
                             add=True)
            if with_deg:
                pltpu.async_copy(ones_v, deg_sh.at[idx_d.at[row]], dsem[b],
                                 add=True)
        for b in range(NBUF):
            pltpu.make_async_copy(rows[b], acc_sh.at[idx_d.at[0]],
                                  ssem[b]).wait()
            if with_deg:
                pltpu.make_async_copy(ones_v, deg_sh.at[idx_d.at[0]],
                                      dsem[b]).wait()
            nrow = jnp.minimum((j + 1) * NBUF + b, CHUNKS - 1)
            start_gather(b, nrow)
        return carry

    lax.fori_loop(0, CHUNKS // NBUF, round_, 0)
    for b in range(NBUF):
        wait_gather(b)
    plsc.subcore_barrier()

    pltpu.sync_copy(acc_sh.at[pl.ds(nbase, NPT)],
                    agg_out.at[c, pl.ds(nbase, NPT)])
    if with_deg:
        pltpu.sync_copy(deg_sh.at[pl.ds(nbase, NPT)],
                        deg_out.at[c, pl.ds(nbase, NPT)])


_sc_agg_deg = pl.kernel(
    functools.partial(_sc_agg_body, True),
    out_type=(jax.ShapeDtypeStruct((NC, N_PAD, D_H), jnp.float32),
              jax.ShapeDtypeStruct((NC, N_PAD, 16), jnp.float32)),
    mesh=_MESH,
    compiler_params=_SC_PARAMS,
    scratch_types=(
        [pltpu.VMEM((CHUNKS, K), jnp.int32),
         pltpu.VMEM((CHUNKS, K), jnp.int32),
         pltpu.VMEM((K, 16), jnp.float32),
         pltpu.VMEM_SHARED((N_PAD, D_H), jnp.float32),
         pltpu.VMEM_SHARED((N_PAD, 16), jnp.float32)]
        + [pltpu.VMEM((K, D_H), jnp.float32)] * NBUF
        + [pltpu.SemaphoreType.DMA] * (3 * NBUF)
    ),
)

_sc_agg = pl.kernel(
    functools.partial(_sc_agg_body, False),
    out_type=jax.ShapeDtypeStruct((NC, N_PAD, D_H), jnp.float32),
    mesh=_MESH,
    compiler_params=_SC_PARAMS,
    scratch_types=(
        [pltpu.VMEM((CHUNKS, K), jnp.int32),
         pltpu.VMEM((CHUNKS, K), jnp.int32),
         pltpu.VMEM_SHARED((N_PAD, D_H), jnp.float32)]
        + [pltpu.VMEM((K, D_H), jnp.float32)] * NBUF
        + [pltpu.SemaphoreType.DMA] * (2 * NBUF)
    ),
)


_BN = 1000


def _mm_body(x_ref, w_ref, b_ref, o_ref):
    o_ref[...] = (jnp.dot(x_ref[...], w_ref[...],
                          preferred_element_type=jnp.float32) + b_ref[...])


def _mm_nb_body(x_ref, w_ref, o_ref):
    o_ref[...] = jnp.dot(x_ref[...], w_ref[...],
                         preferred_element_type=jnp.float32)


def _tc2_body(s1_ref, agg_ref, deg_ref, wn_ref, h_ref, y2_ref):
    agg = agg_ref[0] + agg_ref[1]
    deg = jnp.maximum(deg_ref[0][:, 0:1] + deg_ref[1][:, 0:1], 1.0)
    h = jnp.maximum(s1_ref[...] + agg / deg, 0.0)
    h_ref[...] = h
    y2_ref[...] = jnp.dot(h, wn_ref[...], preferred_element_type=jnp.float32)


def _tc3_body(s2_ref, agg_ref, deg_ref, out_ref):
    agg = agg_ref[0] + agg_ref[1]
    deg = jnp.maximum(deg_ref[0][:, 0:1] + deg_ref[1][:, 0:1], 1.0)
    out_ref[...] = s2_ref[...] + agg / deg


def _row_spec(d):
    return pl.BlockSpec((_BN, d), lambda i: (i, 0))


def _part_spec(d):
    return pl.BlockSpec((NC, _BN, d), lambda i: (0, i, 0))


def _full_spec(r, c):
    return pl.BlockSpec((r, c), lambda i: (0, 0))


def _mm(d_in):
    return pl.pallas_call(
        _mm_body,
        grid=(N // _BN,),
        in_specs=[_row_spec(d_in), _full_spec(d_in, D_H),
                  pl.BlockSpec((D_H,), lambda i: (0,))],
        out_specs=_row_spec(D_H),
        out_shape=jax.ShapeDtypeStruct((N, D_H), jnp.float32),
    )


_mm128 = _mm(D_IN)
_mm64 = _mm(D_H)

_mm128_nb = pl.pallas_call(
    _mm_nb_body,
    grid=(N // _BN,),
    in_specs=[_row_spec(D_IN), _full_spec(D_IN, D_H)],
    out_specs=_row_spec(D_H),
    out_shape=jax.ShapeDtypeStruct((N, D_H), jnp.float32),
)

_tc2 = pl.pallas_call(
    _tc2_body,
    grid=(N // _BN,),
    in_specs=[_row_spec(D_H), _part_spec(D_H), _part_spec(16),
              _full_spec(D_H, D_H)],
    out_specs=[_row_spec(D_H), _row_spec(D_H)],
    out_shape=[jax.ShapeDtypeStruct((N, D_H), jnp.float32)] * 2,
)

_tc3 = pl.pallas_call(
    _tc3_body,
    grid=(N // _BN,),
    in_specs=[_row_spec(D_H), _part_spec(D_H), _part_spec(16)],
    out_specs=_row_spec(D_H),
    out_shape=jax.ShapeDtypeStruct((N, D_H), jnp.float32),
)


def kernel(x, edge_index, W_self1, W_neigh1, b1, W_self2, W_neigh2, b2):
    src = edge_index[0].astype(jnp.int32).reshape(NW, CHUNKS, K)
    dst = edge_index[1].astype(jnp.int32).reshape(NW, CHUNKS, K)
    z64 = jnp.zeros((N_PAD, D_H), jnp.float32)
    z16 = jnp.zeros((N_PAD, 16), jnp.float32)
    ones = jnp.ones((K, 16), jnp.float32)

    y1 = _mm128_nb(x, W_neigh1)
    agg1, deg = _sc_agg_deg(y1, src, dst, z64, z16, ones)
    s1 = _mm128(x, W_self1, b1)
    h, y2 = _tc2(s1, agg1, deg, W_neigh2)
    agg2 = _sc_agg(y2, src, dst, z64)
    s2 = _mm64(h, W_self2, b2)
    return _tc3(s2, agg2, deg)
